# Initial kernel scaffold; baseline (speedup 1.0000x reference)
import jax
import jax.numpy as jnp
from jax.experimental import pallas as pl


def kernel(x, edge_attr, params, edge_index, batch):
    raise NotImplementedError("write your pallas kernel here")


# speedup vs baseline: 1.0024x; 1.0024x over previous
import functools
import numpy as np
import jax
import jax.numpy as jnp
from jax.experimental import pallas as pl
from jax.experimental.pallas import tpu as pltpu

N_NODES = 50000
N_EDGES = 800000
D = 64
N_HEADS = 4
N_LAYERS = 6
KAN_IN = 8
N_GRAPHS = 500
NPG = 100
GRID = 3
K_SPL = 3


def _bn(h, g, b):
    mu = h.mean(axis=0, keepdims=True)
    var = h.var(axis=0, keepdims=True)
    return g * (h - mu) / jnp.sqrt(var + 1e-5) + b


def _gps_layer(h, e, src, dst, lp):
    Ax = h @ lp['A_w'] + lp['A_b']
    Bx = h @ lp['B_w'] + lp['B_b']
    Dx = h @ lp['Dm_w'] + lp['Dm_b']
    Ex = h @ lp['Em_w'] + lp['Em_b']
    Ce = e @ lp['C_w'] + lp['C_b']
    e_hat = Dx[src] + Ex[dst] + Ce
    sig = jax.nn.sigmoid(e_hat)
    num = jax.ops.segment_sum(sig * Bx[src], dst, num_segments=N_NODES)
    den = jax.ops.segment_sum(sig, dst, num_segments=N_NODES)
    h_local = Ax + num / (den + 1e-6)
    h_local = h + jax.nn.relu(_bn(h_local, lp['bn_x_g'], lp['bn_x_b']))
    e_new = e + jax.nn.relu(_bn(e_hat, lp['bn_e_g'], lp['bn_e_b']))
    dh = D // N_HEADS
    q = (h @ lp['Wq_w'] + lp['Wq_b']).reshape(N_GRAPHS, NPG, N_HEADS, dh)
    k = (h @ lp['Wk_w'] + lp['Wk_b']).reshape(N_GRAPHS, NPG, N_HEADS, dh)
    v = (h @ lp['Wv_w'] + lp['Wv_b']).reshape(N_GRAPHS, NPG, N_HEADS, dh)
    scores = jnp.einsum('gqhd,gkhd->ghqk', q, k) / np.sqrt(dh)
    attn = jax.nn.softmax(scores, axis=-1)
    out = jnp.einsum('ghqk,gkhd->gqhd', attn, v).reshape(N_NODES, D)
    out = out @ lp['Wo_w'] + lp['Wo_b']
    h_attn = _bn(h + out, lp['bn_a_g'], lp['bn_a_b'])
    hh = h_local + h_attn
    ff = jax.nn.relu(hh @ lp['W1'] + lp['b1']) @ lp['W2'] + lp['b2']
    hh = _bn(hh + ff, lp['bn_f_g'], lp['bn_f_b'])
    return hh, e_new


def _bspline_basis(x):
    h = 2.0 / GRID
    grid = -1.0 - K_SPL * h + h * jnp.arange(GRID + 2 * K_SPL + 1, dtype=jnp.float32)
    xx = x[..., None]
    b = jnp.where((xx >= grid[:-1]) & (xx < grid[1:]), 1.0, 0.0)
    for p in range(1, K_SPL + 1):
        left = (xx - grid[: -(p + 1)]) / (grid[p:-1] - grid[: -(p + 1)])
        right = (grid[p + 1:] - xx) / (grid[p + 1:] - grid[1:-p])
        b = left * b[..., :-1] + right * b[..., 1:]
    return b


def _kan_layer(x, coef, sb, sp):
    basis = _bspline_basis(x)
    spline = jnp.einsum('bij,ioj->bio', basis, coef)
    base = jax.nn.silu(x)
    post = sb[None] * base[:, :, None] + sp[None] * spline
    return post.sum(axis=1)


def _softplus_body(c_ref, out_ref):
    out_ref[...] = jax.nn.softplus(c_ref[...])


def kernel(x, edge_attr, params, edge_index, batch):
    src = edge_index[0]
    dst = edge_index[1]
    h = jnp.tanh(x @ params['node_emb_w'] + params['node_emb_b'])
    e = jnp.tanh(edge_attr @ params['edge_emb_w'] + params['edge_emb_b'])
    for lp in params['layers']:
        h, e = _gps_layer(h, e, src, dst, lp)
    h = h @ params['scale_w'] + params['scale_b']
    pooled = h.reshape(N_GRAPHS, NPG, KAN_IN).sum(axis=1)
    c = _kan_layer(pooled, params['kan1_coef'], params['kan1_sb'], params['kan1_sp'])
    c = _kan_layer(c, params['kan2_coef'], params['kan2_sb'], params['kan2_sp'])
    out = pl.pallas_call(
        _softplus_body,
        out_shape=jax.ShapeDtypeStruct((N_GRAPHS, 1), jnp.float32),
    )(c)
    return out[:, 0]


# speedup vs baseline: 1.8864x; 1.8820x over previous
import functools
import numpy as np
import jax
import jax.numpy as jnp
from jax import lax
from jax.experimental import pallas as pl
from jax.experimental.pallas import tpu as pltpu
from jax.experimental.pallas import tpu_sc as plsc

N_NODES = 50000
N_EDGES = 800000
D = 64
N_HEADS = 4
N_LAYERS = 6
KAN_IN = 8
N_GRAPHS = 500
NPG = 100
GRID = 3
K_SPL = 3

_SC_INFO = plsc.get_sparse_core_info()
NC, NS = _SC_INFO.num_cores, _SC_INFO.num_subcores
NW = NC * NS
EPW = N_EDGES // NW
CH = 200
NCHUNK = EPW // CH


def _sc_edge_gather_body(dxbx, exp_t, ce, src, dst,
                         eh_o, sig_o, sb_o,
                         srcv, dstv, db, eb, ehb, sgb, sbb, sem):
    wid = lax.axis_index("s") * NC + lax.axis_index("c")
    wbase = wid * EPW

    def chunk(k, _):
        base = wbase + k * CH
        pltpu.sync_copy(src.at[pl.ds(base, CH)], srcv)
        pltpu.sync_copy(dst.at[pl.ds(base, CH)], dstv)
        c1 = pltpu.async_copy(dxbx.at[srcv], db, sem)
        c2 = pltpu.async_copy(exp_t.at[dstv], eb, sem)
        pltpu.sync_copy(ce.at[pl.ds(base, CH), :], ehb)
        c1.wait()
        c2.wait()

        def row(i, _):
            for j in range(4):
                sl = pl.ds(16 * j, 16)
                sh = pl.ds(64 + 16 * j, 16)
                eh = db[i, sl] + eb[i, sl] + ehb[i, sl]
                sg = 1.0 / (1.0 + jnp.exp(-eh))
                ehb[i, sl] = eh
                sgb[i, sl] = sg
                sbb[i, sl] = sg * db[i, sh]
            return 0

        lax.fori_loop(0, CH, row, 0)
        pltpu.sync_copy(ehb, eh_o.at[pl.ds(base, CH), :])
        pltpu.sync_copy(sgb, sig_o.at[pl.ds(base, CH), :])
        pltpu.sync_copy(sbb, sb_o.at[pl.ds(base, CH), :])
        return 0

    lax.fori_loop(0, NCHUNK, chunk, 0)


def _sc_edge_gather(dxbx, exp_t, ce, src, dst):
    f32 = jnp.float32
    out = jax.ShapeDtypeStruct((N_EDGES, D), f32)
    return pl.kernel(
        _sc_edge_gather_body,
        mesh=plsc.VectorSubcoreMesh(core_axis_name="c", subcore_axis_name="s"),
        out_type=[out, out, out],
        scratch_types=[
            pltpu.VMEM((CH,), jnp.int32),
            pltpu.VMEM((CH,), jnp.int32),
            pltpu.VMEM((CH, 2 * D), f32),
            pltpu.VMEM((CH, 2 * D), f32),
            pltpu.VMEM((CH, D), f32),
            pltpu.VMEM((CH, D), f32),
            pltpu.VMEM((CH, D), f32),
            pltpu.SemaphoreType.DMA,
        ],
    )(dxbx, exp_t, ce, src, dst)


def _bn(h, g, b):
    mu = h.mean(axis=0, keepdims=True)
    var = h.var(axis=0, keepdims=True)
    return g * (h - mu) / jnp.sqrt(var + 1e-5) + b


def _gps_layer(h, e, src, dst, lp):
    Ax = h @ lp['A_w'] + lp['A_b']
    Bx = h @ lp['B_w'] + lp['B_b']
    Dx = h @ lp['Dm_w'] + lp['Dm_b']
    Ex = h @ lp['Em_w'] + lp['Em_b']
    Ce = e @ lp['C_w'] + lp['C_b']
    dxbx = jnp.concatenate([Dx, Bx], axis=1)
    exp_t = jnp.concatenate([Ex, jnp.zeros_like(Ex)], axis=1)
    e_hat, sig, sb = _sc_edge_gather(dxbx, exp_t, Ce, src, dst)
    num = jax.ops.segment_sum(sb, dst, num_segments=N_NODES)
    den = jax.ops.segment_sum(sig, dst, num_segments=N_NODES)
    h_local = Ax + num / (den + 1e-6)
    h_local = h + jax.nn.relu(_bn(h_local, lp['bn_x_g'], lp['bn_x_b']))
    e_new = e + jax.nn.relu(_bn(e_hat, lp['bn_e_g'], lp['bn_e_b']))
    dh = D // N_HEADS
    q = (h @ lp['Wq_w'] + lp['Wq_b']).reshape(N_GRAPHS, NPG, N_HEADS, dh)
    k = (h @ lp['Wk_w'] + lp['Wk_b']).reshape(N_GRAPHS, NPG, N_HEADS, dh)
    v = (h @ lp['Wv_w'] + lp['Wv_b']).reshape(N_GRAPHS, NPG, N_HEADS, dh)
    scores = jnp.einsum('gqhd,gkhd->ghqk', q, k) / np.sqrt(dh)
    attn = jax.nn.softmax(scores, axis=-1)
    out = jnp.einsum('ghqk,gkhd->gqhd', attn, v).reshape(N_NODES, D)
    out = out @ lp['Wo_w'] + lp['Wo_b']
    h_attn = _bn(h + out, lp['bn_a_g'], lp['bn_a_b'])
    hh = h_local + h_attn
    ff = jax.nn.relu(hh @ lp['W1'] + lp['b1']) @ lp['W2'] + lp['b2']
    hh = _bn(hh + ff, lp['bn_f_g'], lp['bn_f_b'])
    return hh, e_new


def _bspline_basis(x):
    h = 2.0 / GRID
    grid = -1.0 - K_SPL * h + h * jnp.arange(GRID + 2 * K_SPL + 1, dtype=jnp.float32)
    xx = x[..., None]
    b = jnp.where((xx >= grid[:-1]) & (xx < grid[1:]), 1.0, 0.0)
    for p in range(1, K_SPL + 1):
        left = (xx - grid[: -(p + 1)]) / (grid[p:-1] - grid[: -(p + 1)])
        right = (grid[p + 1:] - xx) / (grid[p + 1:] - grid[1:-p])
        b = left * b[..., :-1] + right * b[..., 1:]
    return b


def _kan_layer(x, coef, sb, sp):
    basis = _bspline_basis(x)
    spline = jnp.einsum('bij,ioj->bio', basis, coef)
    base = jax.nn.silu(x)
    post = sb[None] * base[:, :, None] + sp[None] * spline
    return post.sum(axis=1)


def _softplus_body(c_ref, out_ref):
    out_ref[...] = jax.nn.softplus(c_ref[...])


def kernel(x, edge_attr, params, edge_index, batch):
    src = edge_index[0]
    dst = edge_index[1]
    h = jnp.tanh(x @ params['node_emb_w'] + params['node_emb_b'])
    e = jnp.tanh(edge_attr @ params['edge_emb_w'] + params['edge_emb_b'])
    for lp in params['layers']:
        h, e = _gps_layer(h, e, src, dst, lp)
    h = h @ params['scale_w'] + params['scale_b']
    pooled = h.reshape(N_GRAPHS, NPG, KAN_IN).sum(axis=1)
    c = _kan_layer(pooled, params['kan1_coef'], params['kan1_sb'], params['kan1_sp'])
    c = _kan_layer(c, params['kan2_coef'], params['kan2_sb'], params['kan2_sp'])
    out = pl.pallas_call(
        _softplus_body,
        out_shape=jax.ShapeDtypeStruct((N_GRAPHS, 1), jnp.float32),
    )(c)
    return out[:, 0]


# speedup vs baseline: 1.9103x; 1.0127x over previous
import functools
import numpy as np
import jax
import jax.numpy as jnp
from jax import lax
from jax.experimental import pallas as pl
from jax.experimental.pallas import tpu as pltpu
from jax.experimental.pallas import tpu_sc as plsc

N_NODES = 50000
N_EDGES = 800000
D = 64
N_HEADS = 4
N_LAYERS = 6
KAN_IN = 8
N_GRAPHS = 500
NPG = 100
GRID = 3
K_SPL = 3

_SC_INFO = plsc.get_sparse_core_info()
NC, NS = _SC_INFO.num_cores, _SC_INFO.num_subcores
NW = NC * NS
EPW = N_EDGES // NW
CH = 200
NCHUNK = EPW // CH


def _sc_edge_gather_body(dxbx, exp_t, ce, src, dst,
                         eh_o, sbsg_o,
                         srcv, dstv, db, eb, ehb, sbsg, sem):
    wid = lax.axis_index("s") * NC + lax.axis_index("c")
    wbase = wid * EPW

    def chunk(k, _):
        base = wbase + k * CH
        pltpu.sync_copy(src.at[pl.ds(base, CH)], srcv)
        pltpu.sync_copy(dst.at[pl.ds(base, CH)], dstv)
        c1 = pltpu.async_copy(dxbx.at[srcv], db, sem)
        c2 = pltpu.async_copy(exp_t.at[dstv], eb, sem)
        pltpu.sync_copy(ce.at[pl.ds(base, CH), :], ehb)
        c1.wait()
        c2.wait()

        def row(i, _):
            for j in range(4):
                sl = pl.ds(16 * j, 16)
                sh = pl.ds(64 + 16 * j, 16)
                eh = db[i, sl] + eb[i, sl] + ehb[i, sl]
                sg = 1.0 / (1.0 + jnp.exp(-eh))
                ehb[i, sl] = eh
                sbsg[i, sl] = sg * db[i, sh]
                sbsg[i, sh] = sg
            return 0

        lax.fori_loop(0, CH, row, 0)
        pltpu.sync_copy(ehb, eh_o.at[pl.ds(base, CH), :])
        pltpu.sync_copy(sbsg, sbsg_o.at[pl.ds(base, CH), :])
        return 0

    lax.fori_loop(0, NCHUNK, chunk, 0)


def _sc_edge_gather(dxbx, exp_t, ce, src, dst):
    f32 = jnp.float32
    return pl.kernel(
        _sc_edge_gather_body,
        mesh=plsc.VectorSubcoreMesh(core_axis_name="c", subcore_axis_name="s"),
        out_type=[jax.ShapeDtypeStruct((N_EDGES, D), f32),
                  jax.ShapeDtypeStruct((N_EDGES, 2 * D), f32)],
        scratch_types=[
            pltpu.VMEM((CH,), jnp.int32),
            pltpu.VMEM((CH,), jnp.int32),
            pltpu.VMEM((CH, 2 * D), f32),
            pltpu.VMEM((CH, 2 * D), f32),
            pltpu.VMEM((CH, D), f32),
            pltpu.VMEM((CH, 2 * D), f32),
            pltpu.SemaphoreType.DMA,
        ],
    )(dxbx, exp_t, ce, src, dst)


NQ = 5
QN = N_NODES // NQ
ACCN = 10240
RPT = ACCN // NS
TRASH = QN
SCH = 128
_EPT_BIG = 50048
_NCH_BIG = _EPT_BIG // SCH
_NCH_LAST = (N_EDGES - 15 * _EPT_BIG) // SCH


def _sc_scatter_body(sbsg, idxq, nd_o, zbuf, idxv, rows, acc, sem):
    c = lax.axis_index("c")
    s = lax.axis_index("s")
    tbase = s * _EPT_BIG
    nchunks = jnp.where(s == NS - 1, _NCH_LAST, _NCH_BIG)

    def zrow(i, _):
        for j in range(8):
            zbuf[i, pl.ds(16 * j, 16)] = jnp.zeros((16,), jnp.float32)
        return 0

    lax.fori_loop(0, 200, zrow, 0)

    for r in range(3):
        q = c * 3 + r

        @pl.when(jnp.logical_or(c == 0, r < 2))
        def _round():
            for z in range(3):
                pltpu.sync_copy(zbuf, acc.at[pl.ds(s * RPT + z * 200, 200), :])
            pltpu.sync_copy(zbuf.at[pl.ds(0, RPT - 600), :],
                            acc.at[pl.ds(s * RPT + 600, RPT - 600), :])
            plsc.subcore_barrier()

            def chunk(k, _):
                base = tbase + k * SCH
                pltpu.sync_copy(idxq.at[q, pl.ds(base, SCH)], idxv)
                pltpu.sync_copy(sbsg.at[pl.ds(base, SCH), :], rows)
                pltpu.sync_copy(rows, acc.at[idxv], add=True)
                return 0

            lax.fori_loop(0, nchunks, chunk, 0)
            plsc.subcore_barrier()
            pltpu.sync_copy(acc.at[pl.ds(s * RPT, RPT), :],
                            nd_o.at[q, pl.ds(s * RPT, RPT), :])
            plsc.subcore_barrier()


def _sc_scatter(sbsg, idxq):
    f32 = jnp.float32
    return pl.kernel(
        _sc_scatter_body,
        mesh=plsc.VectorSubcoreMesh(core_axis_name="c", subcore_axis_name="s"),
        out_type=jax.ShapeDtypeStruct((NQ, ACCN, 2 * D), f32),
        scratch_types=[
            pltpu.VMEM((200, 2 * D), f32),
            pltpu.VMEM((SCH,), jnp.int32),
            pltpu.VMEM((SCH, 2 * D), f32),
            pltpu.VMEM_SHARED((ACCN, 2 * D), f32),
            pltpu.SemaphoreType.DMA,
        ],
    )(sbsg, idxq)


def _bn(h, g, b):
    mu = h.mean(axis=0, keepdims=True)
    var = h.var(axis=0, keepdims=True)
    return g * (h - mu) / jnp.sqrt(var + 1e-5) + b


def _gps_layer(h, e, src, dst, idxq, lp):
    Ax = h @ lp['A_w'] + lp['A_b']
    Bx = h @ lp['B_w'] + lp['B_b']
    Dx = h @ lp['Dm_w'] + lp['Dm_b']
    Ex = h @ lp['Em_w'] + lp['Em_b']
    Ce = e @ lp['C_w'] + lp['C_b']
    dxbx = jnp.concatenate([Dx, Bx], axis=1)
    exp_t = jnp.concatenate([Ex, jnp.zeros_like(Ex)], axis=1)
    e_hat, sbsg = _sc_edge_gather(dxbx, exp_t, Ce, src, dst)
    nd = _sc_scatter(sbsg, idxq)[:, :QN, :].reshape(N_NODES, 2 * D)
    num = nd[:, :D]
    den = nd[:, D:]
    h_local = Ax + num / (den + 1e-6)
    h_local = h + jax.nn.relu(_bn(h_local, lp['bn_x_g'], lp['bn_x_b']))
    e_new = e + jax.nn.relu(_bn(e_hat, lp['bn_e_g'], lp['bn_e_b']))
    dh = D // N_HEADS
    q = (h @ lp['Wq_w'] + lp['Wq_b']).reshape(N_GRAPHS, NPG, N_HEADS, dh)
    k = (h @ lp['Wk_w'] + lp['Wk_b']).reshape(N_GRAPHS, NPG, N_HEADS, dh)
    v = (h @ lp['Wv_w'] + lp['Wv_b']).reshape(N_GRAPHS, NPG, N_HEADS, dh)
    scores = jnp.einsum('gqhd,gkhd->ghqk', q, k) / np.sqrt(dh)
    attn = jax.nn.softmax(scores, axis=-1)
    out = jnp.einsum('ghqk,gkhd->gqhd', attn, v).reshape(N_NODES, D)
    out = out @ lp['Wo_w'] + lp['Wo_b']
    h_attn = _bn(h + out, lp['bn_a_g'], lp['bn_a_b'])
    hh = h_local + h_attn
    ff = jax.nn.relu(hh @ lp['W1'] + lp['b1']) @ lp['W2'] + lp['b2']
    hh = _bn(hh + ff, lp['bn_f_g'], lp['bn_f_b'])
    return hh, e_new


def _bspline_basis(x):
    h = 2.0 / GRID
    grid = -1.0 - K_SPL * h + h * jnp.arange(GRID + 2 * K_SPL + 1, dtype=jnp.float32)
    xx = x[..., None]
    b = jnp.where((xx >= grid[:-1]) & (xx < grid[1:]), 1.0, 0.0)
    for p in range(1, K_SPL + 1):
        left = (xx - grid[: -(p + 1)]) / (grid[p:-1] - grid[: -(p + 1)])
        right = (grid[p + 1:] - xx) / (grid[p + 1:] - grid[1:-p])
        b = left * b[..., :-1] + right * b[..., 1:]
    return b


def _kan_layer(x, coef, sb, sp):
    basis = _bspline_basis(x)
    spline = jnp.einsum('bij,ioj->bio', basis, coef)
    base = jax.nn.silu(x)
    post = sb[None] * base[:, :, None] + sp[None] * spline
    return post.sum(axis=1)


def _softplus_body(c_ref, out_ref):
    out_ref[...] = jax.nn.softplus(c_ref[...])


def kernel(x, edge_attr, params, edge_index, batch):
    src = edge_index[0]
    dst = edge_index[1]
    qb = (jnp.arange(NQ, dtype=jnp.int32) * QN)[:, None]
    inr = (dst[None, :] >= qb) & (dst[None, :] < qb + QN)
    idxq = jnp.where(inr, dst[None, :] - qb, TRASH).astype(jnp.int32)
    h = jnp.tanh(x @ params['node_emb_w'] + params['node_emb_b'])
    e = jnp.tanh(edge_attr @ params['edge_emb_w'] + params['edge_emb_b'])
    for lp in params['layers']:
        h, e = _gps_layer(h, e, src, dst, idxq, lp)
    h = h @ params['scale_w'] + params['scale_b']
    pooled = h.reshape(N_GRAPHS, NPG, KAN_IN).sum(axis=1)
    c = _kan_layer(pooled, params['kan1_coef'], params['kan1_sb'], params['kan1_sp'])
    c = _kan_layer(c, params['kan2_coef'], params['kan2_sb'], params['kan2_sp'])
    out = pl.pallas_call(
        _softplus_body,
        out_shape=jax.ShapeDtypeStruct((N_GRAPHS, 1), jnp.float32),
    )(c)
    return out[:, 0]


# speedup vs baseline: 2.2629x; 1.1846x over previous
import functools
import numpy as np
import jax
import jax.numpy as jnp
from jax import lax
from jax.experimental import pallas as pl
from jax.experimental.pallas import tpu as pltpu
from jax.experimental.pallas import tpu_sc as plsc

N_NODES = 50000
N_EDGES = 800000
D = 64
N_HEADS = 4
N_LAYERS = 6
KAN_IN = 8
N_GRAPHS = 500
NPG = 100
GRID = 3
K_SPL = 3

_SC_INFO = plsc.get_sparse_core_info()
NC, NS = _SC_INFO.num_cores, _SC_INFO.num_subcores
NW = NC * NS
CH = 128
GPT = 25088
GPT_LAST = N_EDGES - 31 * GPT
GNCH = GPT // CH


def _sc_edge_gather_body(dxbx, exp_t, ce, src, dst,
                         eh_o, sbsg_o, stat_o,
                         srcv0, srcv1, dstv0, dstv1, db, eb, ehb, statb, sem):
    wid = lax.axis_index("s") * NC + lax.axis_index("c")
    wbase = wid * GPT
    nch = jnp.where(wid == NW - 1, GPT_LAST // CH, GPT // CH)
    srcv = (srcv0, srcv1)
    dstv = (dstv0, dstv1)

    def fill(k, b):
        base = wbase + k * CH
        pltpu.sync_copy(src.at[pl.ds(base, CH)], srcv[b])
        pltpu.sync_copy(dst.at[pl.ds(base, CH)], dstv[b])
        pltpu.async_copy(dxbx.at[srcv[b]], db.at[b], sem)
        pltpu.async_copy(exp_t.at[dstv[b]], eb.at[b], sem)
        pltpu.async_copy(ce.at[pl.ds(base, CH), :], ehb.at[b], sem)

    def drain(b):
        pltpu.make_async_copy(dxbx.at[srcv[b]], db.at[b], sem).wait()
        pltpu.make_async_copy(exp_t.at[dstv[b]], eb.at[b], sem).wait()
        pltpu.make_async_copy(ce.at[pl.ds(0, CH), :], ehb.at[b], sem).wait()

    def process(k, b):
        drain(b)

        def row(i, carry):
            acc = list(carry)
            for j in range(4):
                sl = pl.ds(16 * j, 16)
                sh = pl.ds(64 + 16 * j, 16)
                eh = db[b, i, sl] + eb[b, i, sl] + ehb[b, i, sl]
                sg = 1.0 / (1.0 + jnp.exp(-eh))
                ehb[b, i, sl] = eh
                sb = sg * db[b, i, sh]
                db[b, i, sl] = sb
                db[b, i, sh] = sg
                acc[j] = acc[j] + eh
                acc[4 + j] = acc[4 + j] + eh * eh
            return tuple(acc)

        z = jnp.zeros((16,), jnp.float32)
        stats = lax.fori_loop(0, CH, row, (z,) * 8)
        for j in range(8):
            statb[0, pl.ds(16 * j, 16)] = statb[0, pl.ds(16 * j, 16)] + stats[j]
        base = wbase + k * CH
        pltpu.sync_copy(ehb.at[b], eh_o.at[pl.ds(base, CH), :])
        pltpu.sync_copy(db.at[b], sbsg_o.at[pl.ds(base, CH), :])

    def zstat(j):
        statb[0, pl.ds(16 * j, 16)] = jnp.zeros((16,), jnp.float32)

    for j in range(8):
        zstat(j)
    fill(0, 0)
    fill(1, 1)

    def pair(g, _):
        for b in range(2):
            k = 2 * g + b
            process(k, b)

            @pl.when(k + 2 < nch)
            def _():
                fill(k + 2, b)
        return 0

    lax.fori_loop(0, nch // 2, pair, 0)
    pltpu.sync_copy(statb, stat_o.at[pl.ds(wid, 1), :])


def _sc_edge_gather(dxbx, exp_t, ce, src, dst):
    f32 = jnp.float32
    return pl.kernel(
        _sc_edge_gather_body,
        mesh=plsc.VectorSubcoreMesh(core_axis_name="c", subcore_axis_name="s"),
        out_type=[jax.ShapeDtypeStruct((N_EDGES, D), f32),
                  jax.ShapeDtypeStruct((N_EDGES, 2 * D), f32),
                  jax.ShapeDtypeStruct((NW, 2 * D), f32)],
        scratch_types=[
            pltpu.VMEM((CH,), jnp.int32),
            pltpu.VMEM((CH,), jnp.int32),
            pltpu.VMEM((CH,), jnp.int32),
            pltpu.VMEM((CH,), jnp.int32),
            pltpu.VMEM((2, CH, 2 * D), f32),
            pltpu.VMEM((2, CH, 2 * D), f32),
            pltpu.VMEM((2, CH, D), f32),
            pltpu.VMEM((1, 2 * D), f32),
            pltpu.SemaphoreType.DMA,
        ],
    )(dxbx, exp_t, ce, src, dst)


NQ = 6
QN = 8334
ACCN = 8448
RPT = ACCN // NS
TRASH = QN
SCH = 128
_EPT_BIG = 50048
_NCH_BIG = _EPT_BIG // SCH
_NCH_LAST = (N_EDGES - 15 * _EPT_BIG) // SCH


def _sc_scatter_body(sbsg, idxq, nd_o, zbuf, idxv0, idxv1, rows, acc, sem):
    idxv = (idxv0, idxv1)
    c = lax.axis_index("c")
    s = lax.axis_index("s")
    tbase = s * _EPT_BIG
    nchunks = jnp.where(s == NS - 1, _NCH_LAST, _NCH_BIG)

    def zrow(i, _):
        for j in range(8):
            zbuf[i, pl.ds(16 * j, 16)] = jnp.zeros((16,), jnp.float32)
        return 0

    lax.fori_loop(0, 200, zrow, 0)

    for r in range(3):
        q = c * 3 + r
        if True:
            for z in range(2):
                pltpu.sync_copy(zbuf, acc.at[pl.ds(s * RPT + z * 200, 200), :])
            pltpu.sync_copy(zbuf.at[pl.ds(0, RPT - 400), :],
                            acc.at[pl.ds(s * RPT + 400, RPT - 400), :])
            plsc.subcore_barrier()

            def fill(k, b):
                base = tbase + k * SCH
                pltpu.async_copy(idxq.at[q, pl.ds(base, SCH)], idxv[b], sem)
                pltpu.async_copy(sbsg.at[pl.ds(base, SCH), :], rows.at[b], sem)

            def drain(b):
                pltpu.make_async_copy(idxq.at[q, pl.ds(0, SCH)], idxv[b],
                                      sem).wait()
                pltpu.make_async_copy(sbsg.at[pl.ds(0, SCH), :], rows.at[b],
                                      sem).wait()

            fill(0, 0)

            @pl.when(nchunks > 1)
            def _():
                fill(1, 1)

            def pair(g, _):
                for b in range(2):
                    k = 2 * g + b

                    @pl.when(k < nchunks)
                    def _():
                        drain(b)
                        pltpu.sync_copy(rows.at[b], acc.at[idxv[b]], add=True)

                        @pl.when(k + 2 < nchunks)
                        def _():
                            fill(k + 2, b)
                return 0

            lax.fori_loop(0, (nchunks + 1) // 2, pair, 0)
            plsc.subcore_barrier()
            pltpu.sync_copy(acc.at[pl.ds(s * RPT, RPT), :],
                            nd_o.at[q, pl.ds(s * RPT, RPT), :])
            plsc.subcore_barrier()


def _sc_scatter(sbsg, idxq):
    f32 = jnp.float32
    return pl.kernel(
        _sc_scatter_body,
        mesh=plsc.VectorSubcoreMesh(core_axis_name="c", subcore_axis_name="s"),
        out_type=jax.ShapeDtypeStruct((NQ, ACCN, 2 * D), f32),
        scratch_types=[
            pltpu.VMEM((200, 2 * D), f32),
            pltpu.VMEM((SCH,), jnp.int32),
            pltpu.VMEM((SCH,), jnp.int32),
            pltpu.VMEM((2, SCH, 2 * D), f32),
            pltpu.VMEM_SHARED((ACCN, 2 * D), f32),
            pltpu.SemaphoreType.DMA,
        ],
    )(sbsg, idxq)


def _bn(h, g, b):
    mu = h.mean(axis=0, keepdims=True)
    var = h.var(axis=0, keepdims=True)
    return g * (h - mu) / jnp.sqrt(var + 1e-5) + b


def _gps_layer(h, e, src, dst, idxq, lp):
    Ax = h @ lp['A_w'] + lp['A_b']
    Bx = h @ lp['B_w'] + lp['B_b']
    Dx = h @ lp['Dm_w'] + lp['Dm_b']
    Ex = h @ lp['Em_w'] + lp['Em_b']
    Ce = e @ lp['C_w'] + lp['C_b']
    dxbx = jnp.concatenate([Dx, Bx], axis=1)
    exp_t = jnp.concatenate([Ex, jnp.zeros_like(Ex)], axis=1)
    e_hat, sbsg, _estat = _sc_edge_gather(dxbx, exp_t, Ce, src, dst)
    nd = _sc_scatter(sbsg, idxq)[:, :QN, :].reshape(NQ * QN, 2 * D)[:N_NODES]
    num = nd[:, :D]
    den = nd[:, D:]
    h_local = Ax + num / (den + 1e-6)
    h_local = h + jax.nn.relu(_bn(h_local, lp['bn_x_g'], lp['bn_x_b']))
    e_new = e + jax.nn.relu(_bn(e_hat, lp['bn_e_g'], lp['bn_e_b']))
    dh = D // N_HEADS
    q = (h @ lp['Wq_w'] + lp['Wq_b']).reshape(N_GRAPHS, NPG, N_HEADS, dh)
    k = (h @ lp['Wk_w'] + lp['Wk_b']).reshape(N_GRAPHS, NPG, N_HEADS, dh)
    v = (h @ lp['Wv_w'] + lp['Wv_b']).reshape(N_GRAPHS, NPG, N_HEADS, dh)
    scores = jnp.einsum('gqhd,gkhd->ghqk', q, k) / np.sqrt(dh)
    attn = jax.nn.softmax(scores, axis=-1)
    out = jnp.einsum('ghqk,gkhd->gqhd', attn, v).reshape(N_NODES, D)
    out = out @ lp['Wo_w'] + lp['Wo_b']
    h_attn = _bn(h + out, lp['bn_a_g'], lp['bn_a_b'])
    hh = h_local + h_attn
    ff = jax.nn.relu(hh @ lp['W1'] + lp['b1']) @ lp['W2'] + lp['b2']
    hh = _bn(hh + ff, lp['bn_f_g'], lp['bn_f_b'])
    return hh, e_new


def _bspline_basis(x):
    h = 2.0 / GRID
    grid = -1.0 - K_SPL * h + h * jnp.arange(GRID + 2 * K_SPL + 1, dtype=jnp.float32)
    xx = x[..., None]
    b = jnp.where((xx >= grid[:-1]) & (xx < grid[1:]), 1.0, 0.0)
    for p in range(1, K_SPL + 1):
        left = (xx - grid[: -(p + 1)]) / (grid[p:-1] - grid[: -(p + 1)])
        right = (grid[p + 1:] - xx) / (grid[p + 1:] - grid[1:-p])
        b = left * b[..., :-1] + right * b[..., 1:]
    return b


def _kan_layer(x, coef, sb, sp):
    basis = _bspline_basis(x)
    spline = jnp.einsum('bij,ioj->bio', basis, coef)
    base = jax.nn.silu(x)
    post = sb[None] * base[:, :, None] + sp[None] * spline
    return post.sum(axis=1)


def _softplus_body(c_ref, out_ref):
    out_ref[...] = jax.nn.softplus(c_ref[...])


def kernel(x, edge_attr, params, edge_index, batch):
    src = edge_index[0]
    dst = edge_index[1]
    qb = (jnp.arange(NQ, dtype=jnp.int32) * QN)[:, None]
    inr = (dst[None, :] >= qb) & (dst[None, :] < qb + QN)
    idxq = jnp.where(inr, dst[None, :] - qb, TRASH).astype(jnp.int32)
    h = jnp.tanh(x @ params['node_emb_w'] + params['node_emb_b'])
    e = jnp.tanh(edge_attr @ params['edge_emb_w'] + params['edge_emb_b'])
    for lp in params['layers']:
        h, e = _gps_layer(h, e, src, dst, idxq, lp)
    h = h @ params['scale_w'] + params['scale_b']
    pooled = h.reshape(N_GRAPHS, NPG, KAN_IN).sum(axis=1)
    c = _kan_layer(pooled, params['kan1_coef'], params['kan1_sb'], params['kan1_sp'])
    c = _kan_layer(c, params['kan2_coef'], params['kan2_sb'], params['kan2_sp'])
    out = pl.pallas_call(
        _softplus_body,
        out_shape=jax.ShapeDtypeStruct((N_GRAPHS, 1), jnp.float32),
    )(c)
    return out[:, 0]


# speedup vs baseline: 2.3035x; 1.0179x over previous
import functools
import numpy as np
import jax
import jax.numpy as jnp
from jax import lax
from jax.experimental import pallas as pl
from jax.experimental.pallas import tpu as pltpu
from jax.experimental.pallas import tpu_sc as plsc

N_NODES = 50000
N_EDGES = 800000
D = 64
N_HEADS = 4
N_LAYERS = 6
KAN_IN = 8
N_GRAPHS = 500
NPG = 100
GRID = 3
K_SPL = 3

try:
    _SC_INFO = plsc.get_sparse_core_info()
    NC, NS = _SC_INFO.num_cores, _SC_INFO.num_subcores
except Exception:
    NC, NS = 2, 16
NW = NC * NS
CH = 128
GPT = 25088
GPT_LAST = N_EDGES - 31 * GPT
GNCH = GPT // CH


def _sc_edge_gather_body(dxbx, exp_t, ce, src, dst,
                         eh_o, sbsg_o, stat_o,
                         srcv0, srcv1, dstv0, dstv1, db, eb, ehb, statb, sem):
    wid = lax.axis_index("s") * NC + lax.axis_index("c")
    wbase = wid * GPT
    nch = jnp.where(wid == NW - 1, GPT_LAST // CH, GPT // CH)
    srcv = (srcv0, srcv1)
    dstv = (dstv0, dstv1)

    def fill(k, b):
        base = wbase + k * CH
        pltpu.sync_copy(src.at[pl.ds(base, CH)], srcv[b])
        pltpu.sync_copy(dst.at[pl.ds(base, CH)], dstv[b])
        pltpu.async_copy(dxbx.at[srcv[b]], db.at[b], sem)
        pltpu.async_copy(exp_t.at[dstv[b]], eb.at[b], sem)
        pltpu.async_copy(ce.at[pl.ds(base, CH), :], ehb.at[b], sem)

    def drain(b):
        pltpu.make_async_copy(dxbx.at[srcv[b]], db.at[b], sem).wait()
        pltpu.make_async_copy(exp_t.at[dstv[b]], eb.at[b], sem).wait()
        pltpu.make_async_copy(ce.at[pl.ds(0, CH), :], ehb.at[b], sem).wait()

    def process(k, b):
        drain(b)

        def row(i, carry):
            acc = list(carry)
            for j in range(4):
                sl = pl.ds(16 * j, 16)
                sh = pl.ds(64 + 16 * j, 16)
                eh = db[b, i, sl] + eb[b, i, sl] + ehb[b, i, sl]
                sg = 1.0 / (1.0 + jnp.exp(-eh))
                ehb[b, i, sl] = eh
                sb = sg * db[b, i, sh]
                db[b, i, sl] = sb
                db[b, i, sh] = sg
                acc[j] = acc[j] + eh
                acc[4 + j] = acc[4 + j] + eh * eh
            return tuple(acc)

        z = jnp.zeros((16,), jnp.float32)
        stats = lax.fori_loop(0, CH, row, (z,) * 8)
        for j in range(8):
            statb[0, pl.ds(16 * j, 16)] = statb[0, pl.ds(16 * j, 16)] + stats[j]
        base = wbase + k * CH
        pltpu.sync_copy(ehb.at[b], eh_o.at[pl.ds(base, CH), :])
        pltpu.sync_copy(db.at[b], sbsg_o.at[pl.ds(base, CH), :])

    def zstat(j):
        statb[0, pl.ds(16 * j, 16)] = jnp.zeros((16,), jnp.float32)

    for j in range(8):
        zstat(j)
    fill(0, 0)
    fill(1, 1)

    def pair(g, _):
        for b in range(2):
            k = 2 * g + b
            process(k, b)

            @pl.when(k + 2 < nch)
            def _():
                fill(k + 2, b)
        return 0

    lax.fori_loop(0, nch // 2, pair, 0)
    pltpu.sync_copy(statb, stat_o.at[pl.ds(wid, 1), :])


def _sc_edge_gather(dxbx, exp_t, ce, src, dst):
    f32 = jnp.float32
    return pl.kernel(
        _sc_edge_gather_body,
        mesh=plsc.VectorSubcoreMesh(core_axis_name="c", subcore_axis_name="s"),
        out_type=[jax.ShapeDtypeStruct((N_EDGES, D), f32),
                  jax.ShapeDtypeStruct((N_EDGES, 2 * D), f32),
                  jax.ShapeDtypeStruct((NW, 2 * D), f32)],
        scratch_types=[
            pltpu.VMEM((CH,), jnp.int32),
            pltpu.VMEM((CH,), jnp.int32),
            pltpu.VMEM((CH,), jnp.int32),
            pltpu.VMEM((CH,), jnp.int32),
            pltpu.VMEM((2, CH, 2 * D), f32),
            pltpu.VMEM((2, CH, 2 * D), f32),
            pltpu.VMEM((2, CH, D), f32),
            pltpu.VMEM((1, 2 * D), f32),
            pltpu.SemaphoreType.DMA,
        ],
    )(dxbx, exp_t, ce, src, dst)


NQ = 6
QN = 8334
ACCN = 8448
RPT = ACCN // NS
TRASH = QN
SCH = 128
_EPT_BIG = 50048
_NCH_BIG = _EPT_BIG // SCH
_NCH_LAST = (N_EDGES - 15 * _EPT_BIG) // SCH


def _sc_scatter_body(sbsg, idxq, nd_o, zbuf, idxv0, idxv1, rows, acc, sem):
    idxv = (idxv0, idxv1)
    c = lax.axis_index("c")
    s = lax.axis_index("s")
    tbase = s * _EPT_BIG
    nchunks = jnp.where(s == NS - 1, _NCH_LAST, _NCH_BIG)

    def zrow(i, _):
        for j in range(8):
            zbuf[i, pl.ds(16 * j, 16)] = jnp.zeros((16,), jnp.float32)
        return 0

    lax.fori_loop(0, 200, zrow, 0)

    for r in range(3):
        q = c * 3 + r
        if True:
            for z in range(2):
                pltpu.sync_copy(zbuf, acc.at[pl.ds(s * RPT + z * 200, 200), :])
            pltpu.sync_copy(zbuf.at[pl.ds(0, RPT - 400), :],
                            acc.at[pl.ds(s * RPT + 400, RPT - 400), :])
            plsc.subcore_barrier()

            def fill(k, b):
                base = tbase + k * SCH
                pltpu.async_copy(idxq.at[q, pl.ds(base, SCH)], idxv[b], sem)
                pltpu.async_copy(sbsg.at[pl.ds(base, SCH), :], rows.at[b], sem)

            def drain(b):
                pltpu.make_async_copy(idxq.at[q, pl.ds(0, SCH)], idxv[b],
                                      sem).wait()
                pltpu.make_async_copy(sbsg.at[pl.ds(0, SCH), :], rows.at[b],
                                      sem).wait()

            fill(0, 0)

            @pl.when(nchunks > 1)
            def _():
                fill(1, 1)

            def pair(g, _):
                for b in range(2):
                    k = 2 * g + b

                    @pl.when(k < nchunks)
                    def _():
                        drain(b)
                        pltpu.sync_copy(rows.at[b], acc.at[idxv[b]], add=True)

                        @pl.when(k + 2 < nchunks)
                        def _():
                            fill(k + 2, b)
                return 0

            lax.fori_loop(0, (nchunks + 1) // 2, pair, 0)
            plsc.subcore_barrier()
            pltpu.sync_copy(acc.at[pl.ds(s * RPT, RPT), :],
                            nd_o.at[q, pl.ds(s * RPT, RPT), :])
            plsc.subcore_barrier()


def _sc_scatter(sbsg, idxq):
    f32 = jnp.float32
    return pl.kernel(
        _sc_scatter_body,
        mesh=plsc.VectorSubcoreMesh(core_axis_name="c", subcore_axis_name="s"),
        out_type=jax.ShapeDtypeStruct((NQ, ACCN, 2 * D), f32),
        scratch_types=[
            pltpu.VMEM((200, 2 * D), f32),
            pltpu.VMEM((SCH,), jnp.int32),
            pltpu.VMEM((SCH,), jnp.int32),
            pltpu.VMEM((2, SCH, 2 * D), f32),
            pltpu.VMEM_SHARED((ACCN, 2 * D), f32),
            pltpu.SemaphoreType.DMA,
        ],
    )(sbsg, idxq)


NBLK = 10
BR = N_NODES // NBLK
EBLK = 80
EBR = N_EDGES // EBLK
_EPS_BN = 1e-5


def _full(shape):
    return pl.BlockSpec(shape, lambda i: tuple(0 for _ in shape))


def _node_emb_body(x_ref, w_ref, b_ref, h_ref):
    h_ref[...] = jnp.tanh(
        jax.lax.dot_general(x_ref[...], w_ref[...], (((1,), (0,)), ((), ())),
                            preferred_element_type=jnp.float32) + b_ref[...])


def _node_emb(x, w, b):
    return pl.pallas_call(
        _node_emb_body,
        grid=(NBLK,),
        in_specs=[pl.BlockSpec((BR, D_NODE_IN := 65), lambda i: (i, 0)),
                  _full((65, D)), _full((1, D))],
        out_specs=pl.BlockSpec((BR, D), lambda i: (i, 0)),
        out_shape=jax.ShapeDtypeStruct((N_NODES, D), jnp.float32),
    )(x, w, b.reshape(1, D))


def _edge_emb_body(ea_ref, w_ref, b_ref, cw_ref, cb_ref, e_ref, ce_ref):
    e = jnp.tanh(
        jax.lax.dot_general(ea_ref[...], w_ref[...], (((1,), (0,)), ((), ())),
                            preferred_element_type=jnp.float32) + b_ref[...])
    e_ref[...] = e
    ce_ref[...] = jax.lax.dot_general(
        e, cw_ref[...], (((1,), (0,)), ((), ())),
        preferred_element_type=jnp.float32) + cb_ref[...]


def _edge_emb(ea, w, b, cw, cb):
    return pl.pallas_call(
        _edge_emb_body,
        grid=(EBLK,),
        in_specs=[pl.BlockSpec((EBR, 13), lambda i: (i, 0)),
                  _full((13, D)), _full((1, D)), _full((D, D)), _full((1, D))],
        out_specs=[pl.BlockSpec((EBR, D), lambda i: (i, 0))] * 2,
        out_shape=[jax.ShapeDtypeStruct((N_EDGES, D), jnp.float32)] * 2,
    )(ea, w, b.reshape(1, D), cw, cb.reshape(1, D))


def _node_linear_body(u_ref, stats_ref, g_ref, bb_ref, w7_ref, b7_ref,
                      h_ref, dxbx_ref, expad_ref, ax_ref, qkv_ref,
                      apply_bn):
    u = u_ref[...]
    if apply_bn:
        s = stats_ref[0, :D] * (1.0 / N_NODES)
        ss = stats_ref[0, D:] * (1.0 / N_NODES)
        var = ss - s * s
        h = g_ref[...] * (u - s[None, :]) * jax.lax.rsqrt(var + _EPS_BN)[None, :] \
            + bb_ref[...]
    else:
        h = u
    h_ref[...] = h
    y = jax.lax.dot_general(h, w7_ref[...], (((1,), (0,)), ((), ())),
                            preferred_element_type=jnp.float32) + b7_ref[...]
    dxbx_ref[...] = y[:, 0:128]
    expad_ref[...] = y[:, 128:256]
    ax_ref[...] = y[:, 256:320]
    qkv_ref[...] = y[:, 320:512]


def _node_linear(u, stats, g, bb, w7, b7, apply_bn):
    f32 = jnp.float32
    body = functools.partial(_node_linear_body, apply_bn=apply_bn)
    return pl.pallas_call(
        body,
        grid=(NBLK,),
        in_specs=[pl.BlockSpec((BR, D), lambda i: (i, 0)),
                  _full((1, 2 * D)), _full((1, D)), _full((1, D)),
                  _full((D, 8 * D)), _full((1, 8 * D))],
        out_specs=[pl.BlockSpec((BR, D), lambda i: (i, 0)),
                   pl.BlockSpec((BR, 2 * D), lambda i: (i, 0)),
                   pl.BlockSpec((BR, 2 * D), lambda i: (i, 0)),
                   pl.BlockSpec((BR, D), lambda i: (i, 0)),
                   pl.BlockSpec((BR, 3 * D), lambda i: (i, 0))],
        out_shape=[jax.ShapeDtypeStruct((N_NODES, D), f32),
                   jax.ShapeDtypeStruct((N_NODES, 2 * D), f32),
                   jax.ShapeDtypeStruct((N_NODES, 2 * D), f32),
                   jax.ShapeDtypeStruct((N_NODES, D), f32),
                   jax.ShapeDtypeStruct((N_NODES, 3 * D), f32)],
    )(u, stats, g.reshape(1, D), bb.reshape(1, D), w7, b7)


GPB = 20
ABLK = N_GRAPHS // GPB


def _attn_body(qkv_ref, r0_ref):
    dh = D // N_HEADS
    inv = 1.0 / np.sqrt(dh)
    for g in range(GPB):
        rows = pl.ds(g * NPG, NPG)
        for hd in range(N_HEADS):
            csl = pl.ds(hd * dh, dh)
            q = qkv_ref[rows, csl]
            k = qkv_ref[rows, pl.ds(D + hd * dh, dh)]
            v = qkv_ref[rows, pl.ds(2 * D + hd * dh, dh)]
            sc = jax.lax.dot_general(q, k, (((1,), (1,)), ((), ())),
                                     preferred_element_type=jnp.float32) * inv
            m = jnp.max(sc, axis=1, keepdims=True)
            p = jnp.exp(sc - m)
            attn = p / jnp.sum(p, axis=1, keepdims=True)
            r0_ref[rows, csl] = jax.lax.dot_general(
                attn, v, (((1,), (0,)), ((), ())),
                preferred_element_type=jnp.float32)


def _attn(qkv):
    return pl.pallas_call(
        _attn_body,
        grid=(ABLK,),
        in_specs=[pl.BlockSpec((GPB * NPG, 3 * D), lambda i: (i, 0))],
        out_specs=pl.BlockSpec((GPB * NPG, D), lambda i: (i, 0)),
        out_shape=jax.ShapeDtypeStruct((N_NODES, D), jnp.float32),
    )(qkv)


def _assemble_body(h_ref, r0_ref, ax_ref, nd_ref, wo_ref, bo_ref,
                   hlp_ref, r_ref, stats_ref, acc):
    i = pl.program_id(0)
    hlp = ax_ref[...] + nd_ref[:, :D] / (nd_ref[:, D:] + 1e-6)
    r = h_ref[...] + jax.lax.dot_general(
        r0_ref[...], wo_ref[...], (((1,), (0,)), ((), ())),
        preferred_element_type=jnp.float32) + bo_ref[...]
    hlp_ref[...] = hlp
    r_ref[...] = r
    part = jnp.concatenate(
        [jnp.sum(hlp, axis=0, keepdims=True),
         jnp.sum(hlp * hlp, axis=0, keepdims=True),
         jnp.sum(r, axis=0, keepdims=True),
         jnp.sum(r * r, axis=0, keepdims=True)], axis=1)

    @pl.when(i == 0)
    def _():
        acc[...] = jnp.zeros_like(acc)

    acc[...] = acc[...] + part
    stats_ref[...] = acc[...]


def _assemble(h, r0, ax, nd, wo, bo):
    f32 = jnp.float32
    return pl.pallas_call(
        _assemble_body,
        grid=(NBLK,),
        in_specs=[pl.BlockSpec((BR, D), lambda i: (i, 0)),
                  pl.BlockSpec((BR, D), lambda i: (i, 0)),
                  pl.BlockSpec((BR, D), lambda i: (i, 0)),
                  pl.BlockSpec((BR, 2 * D), lambda i: (i, 0)),
                  _full((D, D)), _full((1, D))],
        out_specs=[pl.BlockSpec((BR, D), lambda i: (i, 0)),
                   pl.BlockSpec((BR, D), lambda i: (i, 0)),
                   _full((1, 4 * D))],
        out_shape=[jax.ShapeDtypeStruct((N_NODES, D), f32),
                   jax.ShapeDtypeStruct((N_NODES, D), f32),
                   jax.ShapeDtypeStruct((1, 4 * D), f32)],
        scratch_shapes=[pltpu.VMEM((1, 4 * D), f32)],
    )(h, r0, ax, nd, wo, bo.reshape(1, D))


def _ffn_body(h_ref, hlp_ref, r_ref, st_ref, gx_ref, bx_ref, ga_ref, ba_ref,
              w1_ref, b1_ref, w2_ref, b2_ref, u_ref, stats_ref, acc):
    i = pl.program_id(0)
    inv_n = 1.0 / N_NODES
    mx = st_ref[0, 0:D] * inv_n
    vx = st_ref[0, D:2 * D] * inv_n - mx * mx
    ma = st_ref[0, 2 * D:3 * D] * inv_n
    va = st_ref[0, 3 * D:4 * D] * inv_n - ma * ma
    bnx = gx_ref[...] * (hlp_ref[...] - mx[None, :]) * \
        jax.lax.rsqrt(vx + _EPS_BN)[None, :] + bx_ref[...]
    bna = ga_ref[...] * (r_ref[...] - ma[None, :]) * \
        jax.lax.rsqrt(va + _EPS_BN)[None, :] + ba_ref[...]
    hh = h_ref[...] + jnp.maximum(bnx, 0.0) + bna
    t = jnp.maximum(jax.lax.dot_general(
        hh, w1_ref[...], (((1,), (0,)), ((), ())),
        preferred_element_type=jnp.float32) + b1_ref[...], 0.0)
    ff = jax.lax.dot_general(t, w2_ref[...], (((1,), (0,)), ((), ())),
                             preferred_element_type=jnp.float32) + b2_ref[...]
    u = hh + ff
    u_ref[...] = u
    part = jnp.concatenate(
        [jnp.sum(u, axis=0, keepdims=True),
         jnp.sum(u * u, axis=0, keepdims=True)], axis=1)

    @pl.when(i == 0)
    def _():
        acc[...] = jnp.zeros_like(acc)

    acc[...] = acc[...] + part
    stats_ref[...] = acc[...]


def _ffn(h, hlp, r, st, gx, bx, ga, ba, w1, b1, w2, b2):
    f32 = jnp.float32
    return pl.pallas_call(
        _ffn_body,
        grid=(NBLK,),
        in_specs=[pl.BlockSpec((BR, D), lambda i: (i, 0)),
                  pl.BlockSpec((BR, D), lambda i: (i, 0)),
                  pl.BlockSpec((BR, D), lambda i: (i, 0)),
                  _full((1, 4 * D)), _full((1, D)), _full((1, D)),
                  _full((1, D)), _full((1, D)),
                  _full((D, 2 * D)), _full((1, 2 * D)),
                  _full((2 * D, D)), _full((1, D))],
        out_specs=[pl.BlockSpec((BR, D), lambda i: (i, 0)),
                   _full((1, 2 * D))],
        out_shape=[jax.ShapeDtypeStruct((N_NODES, D), f32),
                   jax.ShapeDtypeStruct((1, 2 * D), f32)],
        scratch_shapes=[pltpu.VMEM((1, 2 * D), f32)],
    )(h, hlp, r, st, gx.reshape(1, D), bx.reshape(1, D), ga.reshape(1, D),
      ba.reshape(1, D), w1, b1.reshape(1, 2 * D), w2, b2.reshape(1, D))


def _edge_update_body(e_ref, eh_ref, est_ref, ge_ref, be_ref, cw_ref, cb_ref,
                      en_ref, ce_ref):
    inv_e = 1.0 / N_EDGES
    est = jnp.sum(est_ref[...], axis=0)
    mu = est[0:D] * inv_e
    var = est[D:2 * D] * inv_e - mu * mu
    bne = ge_ref[...] * (eh_ref[...] - mu[None, :]) * \
        jax.lax.rsqrt(var + _EPS_BN)[None, :] + be_ref[...]
    en = e_ref[...] + jnp.maximum(bne, 0.0)
    en_ref[...] = en
    ce_ref[...] = jax.lax.dot_general(
        en, cw_ref[...], (((1,), (0,)), ((), ())),
        preferred_element_type=jnp.float32) + cb_ref[...]


def _edge_update(e, eh, est, ge, be, cw, cb):
    f32 = jnp.float32
    return pl.pallas_call(
        _edge_update_body,
        grid=(EBLK,),
        in_specs=[pl.BlockSpec((EBR, D), lambda i: (i, 0)),
                  pl.BlockSpec((EBR, D), lambda i: (i, 0)),
                  _full((NW, 2 * D)), _full((1, D)), _full((1, D)),
                  _full((D, D)), _full((1, D))],
        out_specs=[pl.BlockSpec((EBR, D), lambda i: (i, 0)),
                   pl.BlockSpec((EBR, D), lambda i: (i, 0))],
        out_shape=[jax.ShapeDtypeStruct((N_EDGES, D), f32),
                   jax.ShapeDtypeStruct((N_EDGES, D), f32)],
    )(e, eh, est, ge.reshape(1, D), be.reshape(1, D), cw, cb.reshape(1, D))


def _pool_body(u_ref, st_ref, g_ref, bb_ref, sw_ref, sb_ref, p_ref):
    s = st_ref[0, :D] * (1.0 / N_NODES)
    ss = st_ref[0, D:] * (1.0 / N_NODES)
    var = ss - s * s
    h = g_ref[...] * (u_ref[...] - s[None, :]) * \
        jax.lax.rsqrt(var + _EPS_BN)[None, :] + bb_ref[...]
    hs = jax.lax.dot_general(h, sw_ref[...], (((1,), (0,)), ((), ())),
                             preferred_element_type=jnp.float32) + sb_ref[...]
    rows = jax.lax.broadcasted_iota(jnp.int32, (BR // NPG, BR), 0)
    cols = jax.lax.broadcasted_iota(jnp.int32, (BR // NPG, BR), 1)
    pm = jnp.where(cols // NPG == rows, 1.0, 0.0)
    p_ref[...] = jax.lax.dot_general(
        pm, hs, (((1,), (0,)), ((), ())),
        preferred_element_type=jnp.float32)[None]


def _pool(u, st, g, bb, sw, sb):
    out = pl.pallas_call(
        _pool_body,
        grid=(NBLK,),
        in_specs=[pl.BlockSpec((BR, D), lambda i: (i, 0)),
                  _full((1, 2 * D)), _full((1, D)), _full((1, D)),
                  _full((D, KAN_IN)), _full((1, KAN_IN))],
        out_specs=pl.BlockSpec((1, BR // NPG, KAN_IN), lambda i: (i, 0, 0)),
        out_shape=jax.ShapeDtypeStruct((NBLK, BR // NPG, KAN_IN), jnp.float32),
    )(u, st, g.reshape(1, D), bb.reshape(1, D), sw, sb.reshape(1, KAN_IN))
    return out.reshape(N_GRAPHS, KAN_IN)


NB_SPL = GRID + K_SPL


def _bspline_2d(x):
    hg = 2.0 / GRID
    g0 = -1.0 - K_SPL * hg
    npts = GRID + 2 * K_SPL + 1
    grid_v = [g0 + hg * t for t in range(npts)]
    b = [jnp.where((x >= grid_v[j]) & (x < grid_v[j + 1]), 1.0, 0.0)
         for j in range(npts - 1)]
    for p in range(1, K_SPL + 1):
        nb = len(b) - 1
        b = [(x - grid_v[j]) / (grid_v[j + p] - grid_v[j]) * b[j]
             + (grid_v[j + p + 1] - x) / (grid_v[j + p + 1] - grid_v[j + 1])
             * b[j + 1]
             for j in range(nb)]
    return b


def _kan_apply(x, coef_ref, sb_ref, sp_ref, nin, nout):
    basis = _bspline_2d(x)
    base = x / (1.0 + jnp.exp(-x))
    outs = []
    for o in range(nout):
        spline = basis[0] * coef_ref[pl.ds(o * NB_SPL, 1), :]
        for j in range(1, NB_SPL):
            spline = spline + basis[j] * coef_ref[pl.ds(o * NB_SPL + j, 1), :]
        post = sb_ref[pl.ds(o, 1), :] * base + sp_ref[pl.ds(o, 1), :] * spline
        outs.append(jnp.sum(post, axis=1, keepdims=True))
    return jnp.concatenate(outs, axis=1)


def _head_body(p_ref, c1_ref, sb1_ref, sp1_ref, c2_ref, sb2_ref, sp2_ref,
               out_ref):
    c = _kan_apply(p_ref[...], c1_ref, sb1_ref, sp1_ref, KAN_IN, 5)
    c = _kan_apply(c, c2_ref, sb2_ref, sp2_ref, 5, 1)
    out_ref[...] = jnp.log1p(jnp.exp(-jnp.abs(c))) + jnp.maximum(c, 0.0)


def _head(pooled, p):
    c1 = jnp.transpose(p['kan1_coef'], (1, 2, 0)).reshape(5 * NB_SPL, KAN_IN)
    sb1 = p['kan1_sb'].T
    sp1 = p['kan1_sp'].T
    c2 = jnp.transpose(p['kan2_coef'], (1, 2, 0)).reshape(1 * NB_SPL, 5)
    sb2 = p['kan2_sb'].T
    sp2 = p['kan2_sp'].T
    return pl.pallas_call(
        _head_body,
        out_shape=jax.ShapeDtypeStruct((N_GRAPHS, 1), jnp.float32),
    )(pooled, c1, sb1, sp1, c2, sb2, sp2)


def kernel(x, edge_attr, params, edge_index, batch):
    src = edge_index[0]
    dst = edge_index[1]
    qb = (jnp.arange(NQ, dtype=jnp.int32) * QN)[:, None]
    inr = (dst[None, :] >= qb) & (dst[None, :] < qb + QN)
    idxq = jnp.where(inr, dst[None, :] - qb, TRASH).astype(jnp.int32)

    h0 = _node_emb(x, params['node_emb_w'], params['node_emb_b'])
    lps = params['layers']
    e, ce = _edge_emb(edge_attr, params['edge_emb_w'], params['edge_emb_b'],
                      lps[0]['C_w'], lps[0]['C_b'])

    u = h0
    stats_f = jnp.zeros((1, 2 * D), jnp.float32)
    zb = jnp.zeros((D, D), jnp.float32)
    for li, lp in enumerate(lps):
        w7 = jnp.concatenate(
            [lp['Dm_w'], lp['B_w'], lp['Em_w'], zb, lp['A_w'],
             lp['Wq_w'], lp['Wk_w'], lp['Wv_w']], axis=1)
        b7 = jnp.concatenate(
            [lp['Dm_b'], lp['B_b'], lp['Em_b'], jnp.zeros((D,), jnp.float32),
             lp['A_b'], lp['Wq_b'], lp['Wk_b'], lp['Wv_b']]).reshape(1, 8 * D)
        gprev = lps[li - 1]['bn_f_g'] if li > 0 else params['node_emb_b']
        bprev = lps[li - 1]['bn_f_b'] if li > 0 else params['node_emb_b']
        h, dxbx, expad, ax, qkv = _node_linear(
            u, stats_f, gprev, bprev, w7, b7, apply_bn=(li > 0))
        e_hat, sbsg, estat = _sc_edge_gather(dxbx, expad, ce, src, dst)
        nd = _sc_scatter(sbsg, idxq)[:, :QN, :].reshape(NQ * QN, 2 * D)[:N_NODES]
        r0 = _attn(qkv)
        hlp, r, st4 = _assemble(h, r0, ax, nd, lp['Wo_w'], lp['Wo_b'])
        u, stats_f = _ffn(h, hlp, r, st4, lp['bn_x_g'], lp['bn_x_b'],
                          lp['bn_a_g'], lp['bn_a_b'],
                          lp['W1'], lp['b1'], lp['W2'], lp['b2'])
        if li < N_LAYERS - 1:
            e, ce = _edge_update(e, e_hat, estat, lp['bn_e_g'], lp['bn_e_b'],
                                 lps[li + 1]['C_w'], lps[li + 1]['C_b'])

    pooled = _pool(u, stats_f, lps[-1]['bn_f_g'], lps[-1]['bn_f_b'],
                   params['scale_w'], params['scale_b'])
    return _head(pooled, params)[:, 0]
